# Initial kernel scaffold; baseline (speedup 1.0000x reference)
import functools

import jax
import jax.numpy as jnp
from jax import lax
from jax.experimental import pallas as pl
from jax.experimental.pallas import tpu as pltpu
from jax.experimental.pallas import tpu_sc as plsc

NC = 2
NS = 16
NW = NC * NS
CHUNK = 128
ZCHUNK = 128


def _sc_spmm(h, src_p, dst3, n_chunks, n_nodes, n_pad):
    feat = h.shape[1]
    rows_per_sub_z = n_pad // NS
    rows_per_sub_o = n_nodes // NS
    mesh = plsc.VectorSubcoreMesh(core_axis_name="c", subcore_axis_name="s")

    @functools.partial(
        pl.kernel,
        out_type=jax.ShapeDtypeStruct((NC, n_nodes, feat), jnp.float32),
        mesh=mesh,
        scratch_types=[
            pltpu.VMEM((CHUNK,), jnp.int32),
            pltpu.VMEM((1, CHUNK), jnp.int32),
            pltpu.VMEM((CHUNK, feat), jnp.float32),
            pltpu.VMEM((ZCHUNK, feat), jnp.float32),
            pltpu.VMEM_SHARED((n_pad, feat), jnp.float32),
        ],
    )
    def k(h_hbm, src_hbm, dst_hbm, out_hbm, src_v, dst_v, rows_v, zero_v, agg_sh):
        c = lax.axis_index("c")
        s = lax.axis_index("s")
        wid = s * NC + c

        zf = jnp.zeros((16,), jnp.float32)

        @pl.loop(0, ZCHUNK)
        def _(i):
            @pl.loop(0, feat // 16)
            def _(j):
                zero_v[i, pl.ds(j * 16, 16)] = zf

        @pl.loop(0, rows_per_sub_z // ZCHUNK)
        def _(z):
            pltpu.sync_copy(
                zero_v, agg_sh.at[pl.ds(s * rows_per_sub_z + z * ZCHUNK, ZCHUNK)])

        plsc.subcore_barrier()

        @pl.loop(0, n_chunks)
        def _(i):
            row = wid * n_chunks + i
            pltpu.sync_copy(src_hbm.at[pl.ds(row * CHUNK, CHUNK)], src_v)
            pltpu.sync_copy(dst_hbm.at[row], dst_v)
            pltpu.sync_copy(h_hbm.at[src_v], rows_v)
            pltpu.sync_copy(rows_v, agg_sh.at[dst_v.at[0]], add=True)

        plsc.subcore_barrier()

        pltpu.sync_copy(
            agg_sh.at[pl.ds(s * rows_per_sub_o, rows_per_sub_o)],
            out_hbm.at[c, pl.ds(s * rows_per_sub_o, rows_per_sub_o)])

    return k(h, src_p, dst3)


def _dot(a, b):
    return lax.dot_general(a, b, (((1,), (0,)), ((), ())),
                           precision=lax.Precision.HIGHEST,
                           preferred_element_type=jnp.float32)


def _tc_layer(a, h, w_rel, b_rel, w_root):
    def body(a_ref, h_ref, wr_ref, b_ref, wo_ref, o_ref):
        agg = a_ref[0] + a_ref[1]
        z = _dot(agg, wr_ref[...]) + _dot(h_ref[...], wo_ref[...]) + b_ref[...]
        o_ref[...] = jnp.maximum(z, 0.0)

    return pl.pallas_call(
        body, out_shape=jax.ShapeDtypeStruct(h.shape, jnp.float32),
    )(a, h, w_rel, b_rel.reshape(1, -1), w_root)


def _tc_head(a, h, w_rel, b_rel, w_root, batch_row, wfc1, bfc1, wfc2, bfc2,
             n_groups):
    n_nodes = h.shape[0]
    n_cls = wfc2.shape[1]

    def body(a_ref, h_ref, wr_ref, b_ref, wo_ref, bt_ref, w1_ref, b1_ref,
             w2_ref, b2_ref, o_ref):
        agg = a_ref[0] + a_ref[1]
        z = _dot(agg, wr_ref[...]) + _dot(h_ref[...], wo_ref[...]) + b_ref[...]
        h3 = jnp.maximum(z, 0.0)
        gids = lax.broadcasted_iota(jnp.int32, (n_groups, n_nodes), 0)
        onehot_t = (gids == bt_ref[...]).astype(jnp.float32)
        g = _dot(onehot_t, h3)
        z1 = jnp.maximum(_dot(g, w1_ref[...]) + b1_ref[...], 0.0)
        logits = _dot(z1, w2_ref[...]) + b2_ref[...]
        m = jnp.max(logits, axis=-1, keepdims=True)
        lse = jnp.log(jnp.sum(jnp.exp(logits - m), axis=-1, keepdims=True)) + m
        o_ref[...] = logits - lse

    return pl.pallas_call(
        body, out_shape=jax.ShapeDtypeStruct((n_groups, n_cls), jnp.float32),
    )(a, h, w_rel, b_rel.reshape(1, -1), w_root, batch_row,
      wfc1, bfc1.reshape(1, -1), wfc2, bfc2.reshape(1, -1))


def kernel(x, edge_index, batch, W1_rel, b1_rel, W1_root, W2_rel, b2_rel,
           W2_root, W3_rel, b3_rel, W3_root, W_fc1, b_fc1, W_fc2, b_fc2):
    n_nodes = x.shape[0]
    n_edges = edge_index.shape[1]
    n_groups = W_fc1.shape[0]

    n_chunks = -(-n_edges // (NW * CHUNK))
    e_pad = NW * n_chunks * CHUNK
    pad = e_pad - n_edges
    src_p = jnp.concatenate([edge_index[0], jnp.zeros((pad,), jnp.int32)])
    dst_p = jnp.concatenate(
        [edge_index[1], jnp.full((pad,), n_nodes, jnp.int32)])
    dst3 = dst_p.reshape(-1, 1, CHUNK)

    n_pad = -(-(n_nodes + 1) // (NS * ZCHUNK)) * NS * ZCHUNK

    a1 = _sc_spmm(x, src_p, dst3, n_chunks, n_nodes, n_pad)
    h1 = _tc_layer(a1, x, W1_rel, b1_rel, W1_root)
    a2 = _sc_spmm(h1, src_p, dst3, n_chunks, n_nodes, n_pad)
    h2 = _tc_layer(a2, h1, W2_rel, b2_rel, W2_root)
    a3 = _sc_spmm(h2, src_p, dst3, n_chunks, n_nodes, n_pad)
    batch_row = batch.reshape(1, -1)
    return _tc_head(a3, h2, W3_rel, b3_rel, W3_root, batch_row,
                    W_fc1, b_fc1, W_fc2, b_fc2, n_groups)


# speedup vs baseline: 4.0692x; 4.0692x over previous
import functools

import jax
import jax.numpy as jnp
from jax import lax
from jax.experimental import pallas as pl
from jax.experimental.pallas import tpu as pltpu
from jax.experimental.pallas import tpu_sc as plsc

NC = 2
NS = 16
NW = NC * NS
CHUNK = 128
ZCHUNK = 128


def _sc_spmm(h, src_p, dst3, n_chunks, n_nodes, n_pad):
    feat = h.shape[1]
    rows_per_sub = n_pad // NS
    mesh = plsc.VectorSubcoreMesh(core_axis_name="c", subcore_axis_name="s")

    @functools.partial(
        pl.kernel,
        out_type=jax.ShapeDtypeStruct((NC, n_pad, feat), jnp.float32),
        mesh=mesh,
        scratch_types=[
            pltpu.VMEM((CHUNK,), jnp.int32),
            pltpu.VMEM((1, CHUNK), jnp.int32),
            pltpu.VMEM((CHUNK, feat), jnp.float32),
            pltpu.VMEM((ZCHUNK, feat), jnp.float32),
            pltpu.VMEM_SHARED((n_pad, feat), jnp.float32),
        ],
    )
    def k(h_hbm, src_hbm, dst_hbm, out_hbm, src_v, dst_v, rows_v, zero_v, agg_sh):
        c = lax.axis_index("c")
        s = lax.axis_index("s")
        wid = s * NC + c

        zf = jnp.zeros((16,), jnp.float32)

        @pl.loop(0, ZCHUNK)
        def _(i):
            @pl.loop(0, feat // 16)
            def _(j):
                zero_v[i, pl.ds(j * 16, 16)] = zf

        @pl.loop(0, rows_per_sub // ZCHUNK)
        def _(z):
            pltpu.sync_copy(
                zero_v, agg_sh.at[pl.ds(s * rows_per_sub + z * ZCHUNK, ZCHUNK)])

        plsc.subcore_barrier()

        @pl.loop(0, n_chunks)
        def _(i):
            row = wid * n_chunks + i
            pltpu.sync_copy(src_hbm.at[pl.ds(row * CHUNK, CHUNK)], src_v)
            pltpu.sync_copy(dst_hbm.at[row], dst_v)
            pltpu.sync_copy(h_hbm.at[src_v], rows_v)
            pltpu.sync_copy(rows_v, agg_sh.at[dst_v.at[0]], add=True)

        plsc.subcore_barrier()

        pltpu.sync_copy(
            agg_sh.at[pl.ds(s * rows_per_sub, rows_per_sub)],
            out_hbm.at[c, pl.ds(s * rows_per_sub, rows_per_sub)])

    return k(h, src_p, dst3)


def _dot(a, b):
    return lax.dot_general(a, b, (((1,), (0,)), ((), ())),
                           precision=lax.Precision.HIGHEST,
                           preferred_element_type=jnp.float32)


def _tc_layer(a, h, w_rel, b_rel, w_root):
    n_nodes = h.shape[0]

    def body(a_ref, h_ref, wr_ref, b_ref, wo_ref, o_ref):
        agg = a_ref[0][:n_nodes] + a_ref[1][:n_nodes]
        z = _dot(agg, wr_ref[...]) + _dot(h_ref[...], wo_ref[...]) + b_ref[...]
        o_ref[...] = jnp.maximum(z, 0.0)

    return pl.pallas_call(
        body, out_shape=jax.ShapeDtypeStruct(h.shape, jnp.float32),
    )(a, h, w_rel, b_rel.reshape(1, -1), w_root)


def _tc_head(a, h, w_rel, b_rel, w_root, batch_row, wfc1, bfc1, wfc2, bfc2,
             n_groups):
    n_nodes = h.shape[0]
    n_cls = wfc2.shape[1]

    def body(a_ref, h_ref, wr_ref, b_ref, wo_ref, bt_ref, w1_ref, b1_ref,
             w2_ref, b2_ref, o_ref):
        agg = a_ref[0][:n_nodes] + a_ref[1][:n_nodes]
        z = _dot(agg, wr_ref[...]) + _dot(h_ref[...], wo_ref[...]) + b_ref[...]
        h3 = jnp.maximum(z, 0.0)
        gids = lax.broadcasted_iota(jnp.int32, (n_groups, n_nodes), 0)
        onehot_t = (gids == bt_ref[...]).astype(jnp.float32)
        g = _dot(onehot_t, h3)
        z1 = jnp.maximum(_dot(g, w1_ref[...]) + b1_ref[...], 0.0)
        logits = _dot(z1, w2_ref[...]) + b2_ref[...]
        m = jnp.max(logits, axis=-1, keepdims=True)
        lse = jnp.log(jnp.sum(jnp.exp(logits - m), axis=-1, keepdims=True)) + m
        o_ref[...] = logits - lse

    return pl.pallas_call(
        body, out_shape=jax.ShapeDtypeStruct((n_groups, n_cls), jnp.float32),
    )(a, h, w_rel, b_rel.reshape(1, -1), w_root, batch_row,
      wfc1, bfc1.reshape(1, -1), wfc2, bfc2.reshape(1, -1))


def kernel(x, edge_index, batch, W1_rel, b1_rel, W1_root, W2_rel, b2_rel,
           W2_root, W3_rel, b3_rel, W3_root, W_fc1, b_fc1, W_fc2, b_fc2):
    n_nodes = x.shape[0]
    n_edges = edge_index.shape[1]
    n_groups = W_fc1.shape[0]

    n_chunks = -(-n_edges // (NW * CHUNK))
    e_pad = NW * n_chunks * CHUNK
    pad = e_pad - n_edges
    src_p = jnp.concatenate([edge_index[0], jnp.zeros((pad,), jnp.int32)])
    dst_p = jnp.concatenate(
        [edge_index[1], jnp.full((pad,), n_nodes, jnp.int32)])
    dst3 = dst_p.reshape(-1, 1, CHUNK)

    n_pad = -(-(n_nodes + 1) // (NS * ZCHUNK)) * NS * ZCHUNK

    a1 = _sc_spmm(x, src_p, dst3, n_chunks, n_nodes, n_pad)
    h1 = _tc_layer(a1, x, W1_rel, b1_rel, W1_root)
    a2 = _sc_spmm(h1, src_p, dst3, n_chunks, n_nodes, n_pad)
    h2 = _tc_layer(a2, h1, W2_rel, b2_rel, W2_root)
    a3 = _sc_spmm(h2, src_p, dst3, n_chunks, n_nodes, n_pad)
    batch_row = batch.reshape(1, -1)
    return _tc_head(a3, h2, W3_rel, b3_rel, W3_root, batch_row,
                    W_fc1, b_fc1, W_fc2, b_fc2, n_groups)
